# baseline (speedup 1.0000x reference)
import functools

import jax
import jax.numpy as jnp
from jax import lax
from jax.experimental import pallas as pl
from jax.experimental.pallas import tpu as pltpu
from jax.experimental.pallas import tpu_sc as plsc

VOCAB = 20000
EMBED = 128
OUT = 6
BATCH = 4096
SEQ = 200
LANES = 16
PACK = 128 // LANES
NWORK = 32
BPW = BATCH // NWORK

CB = 8
NCH = BPW // CB
CHROWS = CB * SEQ
NBUF = 3

RPW = 125
NRW = (VOCAB // PACK) // RPW

PROJ_BLK = 250

_MESH = plsc.VectorSubcoreMesh(core_axis_name="c", subcore_axis_name="s")
_SC_PARAMS = pltpu.CompilerParams(use_tc_tiling_on_sc=False)


PROJ_SLABS = 10
SLAB = (VOCAB // PACK) // PROJ_SLABS


def _proj_body(e_hbm, w_ref, b_ref, o_hbm, ev, ov, sin, sout):
    w = jnp.concatenate(
        [w_ref[...], jnp.zeros((LANES - OUT, EMBED), jnp.float32)], axis=0)
    b = jnp.concatenate(
        [b_ref[...], jnp.zeros((1, LANES - OUT), jnp.float32)], axis=1)

    H = SLAB // 2

    def in_start(s, p):
        pltpu.make_async_copy(
            e_hbm.at[pl.ds(s * SLAB, H)], ev.at[p].at[pl.ds(0, H)],
            sin.at[p]).start()
        pltpu.make_async_copy(
            e_hbm.at[pl.ds(s * SLAB + H, H)], ev.at[p].at[pl.ds(H, H)],
            sin.at[p]).start()

    def in_wait(s, p):
        pltpu.make_async_copy(
            e_hbm.at[pl.ds(s * SLAB, SLAB)], ev.at[p], sin.at[p]).wait()

    def out_copy(s, p):
        return pltpu.make_async_copy(
            ov.at[p], o_hbm.at[pl.ds(s * SLAB, SLAB)], sout.at[p])

    in_start(0, 0)
    in_start(1, 1)
    for s in range(PROJ_SLABS):
        p = s % 2
        in_wait(s, p)
        if s >= 2:
            out_copy(s - 2, p).wait()
        for k in range(PACK):
            y = lax.dot_general(
                ev[p, :, k, :], w,
                (((1,), (1,)), ((), ())),
                preferred_element_type=jnp.float32,
            ) + b
            ov[p, :, pl.ds(k * LANES, LANES)] = y
        out_copy(s, p).start()
        if s + 2 < PROJ_SLABS:
            in_start(s + 2, p)
    out_copy(PROJ_SLABS - 2, 0).wait()
    out_copy(PROJ_SLABS - 1, 1).wait()


def _project(e3, fc_weight, fc_bias2):
    n = VOCAB // PACK
    return pl.pallas_call(
        _proj_body,
        in_specs=[
            pl.BlockSpec(memory_space=pl.ANY),
            pl.BlockSpec((OUT, EMBED), lambda: (0, 0)),
            pl.BlockSpec((1, OUT), lambda: (0, 0)),
        ],
        out_specs=pl.BlockSpec(memory_space=pl.ANY),
        out_shape=jax.ShapeDtypeStruct((n, 128), jnp.float32),
        scratch_shapes=[
            pltpu.VMEM((2, SLAB, PACK, EMBED), jnp.float32),
            pltpu.VMEM((2, SLAB, 128), jnp.float32),
            pltpu.SemaphoreType.DMA((2,)),
            pltpu.SemaphoreType.DMA((2,)),
        ],
    )(e3, fc_weight, fc_bias2)


@functools.partial(
    pl.kernel,
    out_type=jax.ShapeDtypeStruct((VOCAB, LANES), jnp.float32),
    mesh=_MESH,
    compiler_params=_SC_PARAMS,
    scratch_types=[
        pltpu.VMEM((RPW, 128), jnp.float32),
        pltpu.VMEM((RPW * PACK, LANES), jnp.float32),
    ],
)
def _sc_repack(p_hbm, out_hbm, in_v, out_v):
    wid = lax.axis_index("c") * 16 + lax.axis_index("s")

    @pl.when(wid < NRW)
    def _():
        r0 = wid * RPW
        pltpu.sync_copy(p_hbm.at[pl.ds(r0, RPW)], in_v)

        def row(r, carry):
            for t in range(PACK):
                out_v[r * PACK + t] = in_v[r, pl.ds(t * LANES, LANES)]
            return carry

        lax.fori_loop(0, RPW, row, 0)
        pltpu.sync_copy(out_v, out_hbm.at[pl.ds(r0 * PACK, RPW * PACK)])


@functools.partial(
    pl.kernel,
    out_type=jax.ShapeDtypeStruct((BATCH // PACK, 128), jnp.float32),
    mesh=_MESH,
    compiler_params=_SC_PARAMS,
    scratch_types=[
        pltpu.VMEM((BPW * SEQ,), jnp.int32),
        pltpu.VMEM((NBUF, CHROWS, LANES), jnp.float32),
        pltpu.VMEM((BPW // PACK, 128), jnp.float32),
        pltpu.SemaphoreType.DMA,
        pltpu.SemaphoreType.DMA,
        pltpu.SemaphoreType.DMA,
    ],
)
def _sc_pool(p_hbm, idx_hbm, out_hbm, idx_v, rows_v, ost_v, *sems):
    wid = lax.axis_index("c") * 16 + lax.axis_index("s")
    base = wid * BPW
    pltpu.sync_copy(idx_hbm.at[pl.ds(base * SEQ, BPW * SEQ)], idx_v)

    def issue(c, p):
        pltpu.async_copy(
            p_hbm.at[idx_v.at[pl.ds(c * CHROWS, CHROWS)]],
            rows_v.at[p], sems[p])

    def wait(p):
        pltpu.make_async_copy(
            p_hbm.at[pl.ds(0, CHROWS)], rows_v.at[p], sems[p]).wait()

    def reduce_chunk(c, p):
        for k in range(CB):
            def red(i, accs):
                r0 = k * SEQ + i * 8
                return tuple(accs[t] + rows_v[p, r0 + t] for t in range(8))

            accs = lax.fori_loop(
                0, SEQ // 8, red,
                tuple(jnp.zeros((LANES,), jnp.float32) for _ in range(8)))
            acc = (((accs[0] + accs[1]) + (accs[2] + accs[3]))
                   + ((accs[4] + accs[5]) + (accs[6] + accs[7])))
            ost_v[c, pl.ds(k * LANES, LANES)] = acc * (1.0 / SEQ)

    for p in range(NBUF):
        issue(p, p)

    def ring(h, carry):
        for q in range(NBUF):
            c = h * NBUF + q
            wait(q)
            reduce_chunk(c, q)
            issue(c + NBUF, q)
        return carry

    lax.fori_loop(0, (NCH - NBUF - 1) // NBUF, ring, 0)
    c0 = ((NCH - NBUF - 1) // NBUF) * NBUF
    for c in range(c0, NCH):
        wait(c % NBUF)
        reduce_chunk(c, c % NBUF)
        if c + NBUF < NCH:
            issue(c + NBUF, c % NBUF)

    pltpu.sync_copy(ost_v, out_hbm.at[pl.ds(wid * (BPW // PACK), BPW // PACK)])


def kernel(indices, embed_table, fc_weight, fc_bias):
    e3 = embed_table.reshape(VOCAB // PACK, PACK, EMBED)
    p_packed = _project(e3, fc_weight, fc_bias.reshape(1, OUT))
    p_lin = _sc_repack(p_packed)
    out_packed = _sc_pool(p_lin, indices.reshape(-1))
    out16 = out_packed.reshape(BATCH, LANES)
    return out16[:, :OUT][:, None, :]


# speedup vs baseline: 1.0055x; 1.0055x over previous
import functools

import jax
import jax.numpy as jnp
from jax import lax
from jax.experimental import pallas as pl
from jax.experimental.pallas import tpu as pltpu
from jax.experimental.pallas import tpu_sc as plsc

VOCAB = 20000
EMBED = 128
OUT = 6
BATCH = 4096
SEQ = 200
LANES = 16
PACK = 128 // LANES
NWORK = 32
BPW = BATCH // NWORK

CB = 8
NCH = BPW // CB
CHROWS = CB * SEQ
NBUF = 3

RPW = 125
NRW = (VOCAB // PACK) // RPW

PROJ_BLK = 250

_MESH = plsc.VectorSubcoreMesh(core_axis_name="c", subcore_axis_name="s")
_SC_PARAMS = pltpu.CompilerParams(use_tc_tiling_on_sc=False)


def _proj_body(e_ref, w_ref, b_ref, o_ref):
    w = jnp.concatenate(
        [w_ref[...], jnp.zeros((LANES - OUT, EMBED), jnp.float32)], axis=0)
    b = jnp.concatenate(
        [b_ref[...], jnp.zeros((1, LANES - OUT), jnp.float32)], axis=1)
    for k in range(PACK):
        y = lax.dot_general(
            e_ref[:, k, :], w,
            (((1,), (1,)), ((), ())),
            preferred_element_type=jnp.float32,
        ) + b
        o_ref[:, pl.ds(k * LANES, LANES)] = y


def _project(e3, fc_weight, fc_bias2):
    n = VOCAB // PACK
    return pl.pallas_call(
        _proj_body,
        out_shape=jax.ShapeDtypeStruct((n, 128), jnp.float32),
    )(e3, fc_weight, fc_bias2)


@functools.partial(
    pl.kernel,
    out_type=jax.ShapeDtypeStruct((VOCAB, LANES), jnp.float32),
    mesh=_MESH,
    compiler_params=_SC_PARAMS,
    scratch_types=[
        pltpu.VMEM((RPW, 128), jnp.float32),
        pltpu.VMEM((RPW * PACK, LANES), jnp.float32),
    ],
)
def _sc_repack(p_hbm, out_hbm, in_v, out_v):
    wid = lax.axis_index("c") * 16 + lax.axis_index("s")

    @pl.when(wid < NRW)
    def _():
        r0 = wid * RPW
        pltpu.sync_copy(p_hbm.at[pl.ds(r0, RPW)], in_v)

        def row(r, carry):
            for t in range(PACK):
                out_v[r * PACK + t] = in_v[r, pl.ds(t * LANES, LANES)]
            return carry

        lax.fori_loop(0, RPW, row, 0)
        pltpu.sync_copy(out_v, out_hbm.at[pl.ds(r0 * PACK, RPW * PACK)])


@functools.partial(
    pl.kernel,
    out_type=jax.ShapeDtypeStruct((BATCH // PACK, 128), jnp.float32),
    mesh=_MESH,
    compiler_params=_SC_PARAMS,
    scratch_types=[
        pltpu.VMEM((BPW * SEQ,), jnp.int32),
        pltpu.VMEM((NBUF, CHROWS, LANES), jnp.float32),
        pltpu.VMEM((BPW // PACK, 128), jnp.float32),
        pltpu.SemaphoreType.DMA,
        pltpu.SemaphoreType.DMA,
        pltpu.SemaphoreType.DMA,
    ],
)
def _sc_pool(p_hbm, idx_hbm, out_hbm, idx_v, rows_v, ost_v, *sems):
    wid = lax.axis_index("c") * 16 + lax.axis_index("s")
    base = wid * BPW
    pltpu.sync_copy(idx_hbm.at[pl.ds(base * SEQ, BPW * SEQ)], idx_v)

    def issue(c, p):
        pltpu.async_copy(
            p_hbm.at[idx_v.at[pl.ds(c * CHROWS, CHROWS)]],
            rows_v.at[p], sems[p])

    def wait(p):
        pltpu.make_async_copy(
            p_hbm.at[pl.ds(0, CHROWS)], rows_v.at[p], sems[p]).wait()

    def reduce_chunk(c, p):
        for k in range(CB):
            def red(i, accs):
                r0 = k * SEQ + i * 8
                return tuple(accs[t] + rows_v[p, r0 + t] for t in range(8))

            accs = lax.fori_loop(
                0, SEQ // 8, red,
                tuple(jnp.zeros((LANES,), jnp.float32) for _ in range(8)))
            acc = (((accs[0] + accs[1]) + (accs[2] + accs[3]))
                   + ((accs[4] + accs[5]) + (accs[6] + accs[7])))
            ost_v[c, pl.ds(k * LANES, LANES)] = acc * (1.0 / SEQ)

    for p in range(NBUF):
        issue(p, p)

    def ring(h, carry):
        for q in range(NBUF):
            c = h * NBUF + q
            wait(q)
            reduce_chunk(c, q)
            issue(c + NBUF, q)
        return carry

    lax.fori_loop(0, (NCH - NBUF - 1) // NBUF, ring, 0)
    c0 = ((NCH - NBUF - 1) // NBUF) * NBUF
    for c in range(c0, NCH):
        wait(c % NBUF)
        reduce_chunk(c, c % NBUF)
        if c + NBUF < NCH:
            issue(c + NBUF, c % NBUF)

    pltpu.sync_copy(ost_v, out_hbm.at[pl.ds(wid * (BPW // PACK), BPW // PACK)])


def kernel(indices, embed_table, fc_weight, fc_bias):
    e3 = embed_table.reshape(VOCAB // PACK, PACK, EMBED)
    p_packed = _project(e3, fc_weight, fc_bias.reshape(1, OUT))
    p_lin = _sc_repack(p_packed)
    out_packed = _sc_pool(p_lin, indices.reshape(-1))
    out16 = out_packed.reshape(BATCH, LANES)
    return out16[:, :OUT][:, None, :]


# speedup vs baseline: 1.0071x; 1.0016x over previous
import functools

import jax
import jax.numpy as jnp
from jax import lax
from jax.experimental import pallas as pl
from jax.experimental.pallas import tpu as pltpu
from jax.experimental.pallas import tpu_sc as plsc

VOCAB = 20000
EMBED = 128
OUT = 6
BATCH = 4096
SEQ = 200
LANES = 16
PACK = 128 // LANES
NWORK = 32
BPW = BATCH // NWORK

CB = 8
NCH = BPW // CB
CHROWS = CB * SEQ
NBUF = 3

RPW = 125
NRW = (VOCAB // PACK) // RPW

_MESH = plsc.VectorSubcoreMesh(core_axis_name="c", subcore_axis_name="s")
_SC_PARAMS = pltpu.CompilerParams(use_tc_tiling_on_sc=False)


def _proj_body(e_ref, w_ref, b_ref, o_ref):
    w = jnp.concatenate(
        [w_ref[...], jnp.zeros((LANES - OUT, EMBED), jnp.float32)], axis=0)
    b = jnp.concatenate(
        [b_ref[...], jnp.zeros((1, LANES - OUT), jnp.float32)], axis=1)
    for k in range(PACK):
        y = lax.dot_general(
            e_ref[:, k, :], w,
            (((1,), (1,)), ((), ())),
            preferred_element_type=jnp.float32,
        ) + b
        o_ref[:, pl.ds(k * LANES, LANES)] = y


def _project(e3, fc_weight, fc_bias2):
    n = VOCAB // PACK
    return pl.pallas_call(
        _proj_body,
        out_shape=jax.ShapeDtypeStruct((n, 128), jnp.float32),
    )(e3, fc_weight, fc_bias2)


@functools.partial(
    pl.kernel,
    out_type=jax.ShapeDtypeStruct((VOCAB, LANES), jnp.float32),
    mesh=_MESH,
    compiler_params=_SC_PARAMS,
    scratch_types=[
        pltpu.VMEM((RPW, 128), jnp.float32),
        pltpu.VMEM((RPW * PACK, LANES), jnp.float32),
    ],
)
def _sc_repack(p_hbm, out_hbm, in_v, out_v):
    wid = lax.axis_index("c") * 16 + lax.axis_index("s")

    @pl.when(wid < NRW)
    def _():
        r0 = wid * RPW
        pltpu.sync_copy(p_hbm.at[pl.ds(r0, RPW)], in_v)

        def row(r, carry):
            for t in range(PACK):
                out_v[r * PACK + t] = in_v[r, pl.ds(t * LANES, LANES)]
            return carry

        lax.fori_loop(0, RPW, row, 0)
        pltpu.sync_copy(out_v, out_hbm.at[pl.ds(r0 * PACK, RPW * PACK)])


@functools.partial(
    pl.kernel,
    out_type=jax.ShapeDtypeStruct((BATCH // PACK, 128), jnp.float32),
    mesh=_MESH,
    compiler_params=_SC_PARAMS,
    scratch_types=[
        pltpu.VMEM((BPW * SEQ,), jnp.int32),
        pltpu.VMEM((NBUF, CHROWS, LANES), jnp.float32),
        pltpu.VMEM((BPW // PACK, 128), jnp.float32),
        pltpu.SemaphoreType.DMA,
        pltpu.SemaphoreType.DMA,
        pltpu.SemaphoreType.DMA,
    ],
)
def _sc_pool(p_hbm, idx_hbm, out_hbm, idx_v, rows_v, ost_v, *sems):
    wid = lax.axis_index("c") * 16 + lax.axis_index("s")
    base = wid * BPW
    pltpu.sync_copy(idx_hbm.at[pl.ds(base * SEQ, BPW * SEQ)], idx_v)

    def issue(c, p):
        pltpu.async_copy(
            p_hbm.at[idx_v.at[pl.ds(c * CHROWS, CHROWS)]],
            rows_v.at[p], sems[p])

    def wait(p):
        pltpu.make_async_copy(
            p_hbm.at[pl.ds(0, CHROWS)], rows_v.at[p], sems[p]).wait()

    def reduce_chunk(c, p):
        for k in range(CB):
            def red(i, accs):
                r0 = k * SEQ + i * 8
                return tuple(accs[t] + rows_v[p, r0 + t] for t in range(8))

            accs = lax.fori_loop(
                0, SEQ // 8, red,
                tuple(jnp.zeros((LANES,), jnp.float32) for _ in range(8)))
            acc = (((accs[0] + accs[1]) + (accs[2] + accs[3]))
                   + ((accs[4] + accs[5]) + (accs[6] + accs[7])))
            ost_v[c, pl.ds(k * LANES, LANES)] = acc * (1.0 / SEQ)

    for p in range(NBUF):
        issue(p, p)

    def ring(h, carry):
        for q in range(NBUF):
            c = h * NBUF + q
            wait(q)
            reduce_chunk(c, q)
            issue(c + NBUF, q)
        return carry

    lax.fori_loop(0, (NCH - NBUF - 1) // NBUF, ring, 0)
    c0 = ((NCH - NBUF - 1) // NBUF) * NBUF
    for c in range(c0, NCH):
        wait(c % NBUF)
        reduce_chunk(c, c % NBUF)
        if c + NBUF < NCH:
            issue(c + NBUF, c % NBUF)

    pltpu.sync_copy(ost_v, out_hbm.at[pl.ds(wid * (BPW // PACK), BPW // PACK)])


def kernel(indices, embed_table, fc_weight, fc_bias):
    e3 = embed_table.reshape(VOCAB // PACK, PACK, EMBED)
    p_packed = _project(e3, fc_weight, fc_bias.reshape(1, OUT))
    p_lin = _sc_repack(p_packed)
    out_packed = _sc_pool(p_lin, indices.reshape(-1))
    out16 = out_packed.reshape(BATCH, LANES)
    return out16[:, :OUT][:, None, :]
